# baseline (speedup 1.0000x reference)
import functools
import math

import jax
import jax.numpy as jnp
from jax import lax
from jax.experimental import pallas as pl
from jax.experimental.pallas import tpu as pltpu
from jax.experimental.pallas import tpu_sc as plsc

D_MODEL = 64
VOCAB = 1000000
B_TOTAL = 4096 * 200
NC, NS = 2, 16
NW = NC * NS
ROWS_PER_W = B_TOTAL // NW
SUB = 4
CHUNK = SUB * 128
N_CHUNKS = ROWS_PER_W // CHUNK
SCALE = math.sqrt(D_MODEL)

_mesh = plsc.VectorSubcoreMesh(
    core_axis_name="c", subcore_axis_name="s", num_cores=NC, num_subcores=NS
)


@functools.partial(
    pl.kernel,
    out_type=jax.ShapeDtypeStruct((B_TOTAL, D_MODEL), jnp.float32),
    mesh=_mesh,
    scratch_types=[
        pltpu.VMEM((SUB, 128), jnp.int32),
        pltpu.VMEM((CHUNK, D_MODEL), jnp.float32),
        pltpu.SemaphoreType.DMA,
    ],
    compiler_params=pltpu.CompilerParams(use_tc_tiling_on_sc=False),
)
def _embed_gather(lut_hbm, idx_hbm, out_hbm, idx_v, rows_v, sem):
    wid = lax.axis_index("s") * NC + lax.axis_index("c")
    irow0 = wid * (ROWS_PER_W // 128)
    row0 = wid * ROWS_PER_W

    def chunk_body(g, carry):
        pltpu.sync_copy(idx_hbm.at[pl.ds(irow0 + g * SUB, SUB)], idx_v)
        copies = []
        for j in range(SUB):
            copies.append(
                pltpu.async_copy(
                    lut_hbm.at[idx_v.at[j]],
                    rows_v.at[pl.ds(j * 128, 128)],
                    sem,
                )
            )
        for c in copies:
            c.wait()

        def scale_row(i, c2):
            for j in range(D_MODEL // 16):
                sl = pl.ds(j * 16, 16)
                rows_v[i, sl] = rows_v[i, sl] * SCALE
            return c2

        lax.fori_loop(0, CHUNK, scale_row, 0)

        pltpu.sync_copy(rows_v, out_hbm.at[pl.ds(row0 + g * CHUNK, CHUNK)])
        return carry

    lax.fori_loop(0, N_CHUNKS, chunk_body, 0)


def kernel(x, lut):
    idx2 = x.reshape(B_TOTAL // 128, 128).astype(jnp.int32)
    out = _embed_gather(lut, idx2)
    return out.reshape(x.shape[0], x.shape[1], D_MODEL)


# speedup vs baseline: 1.1372x; 1.1372x over previous
import functools
import math

import jax
import jax.numpy as jnp
from jax import lax
from jax.experimental import pallas as pl
from jax.experimental.pallas import tpu as pltpu
from jax.experimental.pallas import tpu_sc as plsc

D_MODEL = 64
VOCAB = 1000000
B_TOTAL = 4096 * 200
NC, NS = 2, 16
NW = NC * NS
ROWS_PER_W = B_TOTAL // NW
SUB = 4
CHUNK = SUB * 128
N_CHUNKS = ROWS_PER_W // CHUNK
NBUF = 2
SCALE = math.sqrt(D_MODEL)

_mesh = plsc.VectorSubcoreMesh(
    core_axis_name="c", subcore_axis_name="s", num_cores=NC, num_subcores=NS
)


@functools.partial(
    pl.kernel,
    out_type=jax.ShapeDtypeStruct((B_TOTAL, D_MODEL), jnp.float32),
    mesh=_mesh,
    scratch_types=[
        pltpu.VMEM((NBUF, SUB, 128), jnp.int32),
        pltpu.VMEM((NBUF, CHUNK, D_MODEL), jnp.float32),
        pltpu.SemaphoreType.DMA,
        pltpu.SemaphoreType.DMA,
        pltpu.SemaphoreType.DMA,
        pltpu.SemaphoreType.DMA,
        pltpu.SemaphoreType.DMA,
        pltpu.SemaphoreType.DMA,
    ],
    compiler_params=pltpu.CompilerParams(use_tc_tiling_on_sc=False),
)
def _embed_gather(lut_hbm, idx_hbm, out_hbm, idx_v, rows_v,
                  si0, si1, sg0, sg1, ss0, ss1):
    si = (si0, si1)
    sg = (sg0, sg1)
    ss = (ss0, ss1)
    wid = lax.axis_index("s") * NC + lax.axis_index("c")
    irow0 = wid * (ROWS_PER_W // 128)
    row0 = wid * ROWS_PER_W

    def idx_start(g, b):
        pltpu.async_copy(
            idx_hbm.at[pl.ds(irow0 + g * SUB, SUB)], idx_v.at[b], si[b])

    def idx_wait(b):
        pltpu.make_async_copy(
            idx_hbm.at[pl.ds(irow0, SUB)], idx_v.at[b], si[b]).wait()

    def gat_start(b):
        for j in range(SUB):
            pltpu.async_copy(
                lut_hbm.at[idx_v.at[b, j]],
                rows_v.at[b, pl.ds(j * 128, 128)], sg[b])

    def gat_wait(b):
        for j in range(SUB):
            pltpu.make_async_copy(
                lut_hbm.at[idx_v.at[b, j]],
                rows_v.at[b, pl.ds(j * 128, 128)], sg[b]).wait()

    def store_start(g, b):
        pltpu.async_copy(
            rows_v.at[b], out_hbm.at[pl.ds(row0 + g * CHUNK, CHUNK)], ss[b])

    def store_wait(b):
        pltpu.make_async_copy(
            rows_v.at[b], out_hbm.at[pl.ds(row0, CHUNK)], ss[b]).wait()

    def scale_buf(b):
        rv = rows_v.at[b]

        def scale_blk(k, c):
            base = k * 4
            for u in range(4):
                for j in range(D_MODEL // 16):
                    sl = pl.ds(j * 16, 16)
                    rv[base + u, sl] = rv[base + u, sl] * SCALE
            return c

        lax.fori_loop(0, CHUNK // 4, scale_blk, 0)

    idx_start(0, 0)
    idx_start(1, 1)
    idx_wait(0)
    gat_start(0)

    def step(it, carry):
        for b in range(NBUF):
            g = it * NBUF + b
            o = b ^ 1
            gat_wait(b)
            pl.when(g < N_CHUNKS - NBUF)(lambda: idx_start(g + NBUF, b))
            pl.when(g >= 1)(lambda: store_wait(o))
            def fire_next():
                idx_wait(o)
                gat_start(o)
            pl.when(g < N_CHUNKS - 1)(fire_next)
            scale_buf(b)
            store_start(g, b)
        return carry

    lax.fori_loop(0, N_CHUNKS // NBUF, step, 0)

    store_wait((N_CHUNKS - 1) % NBUF)


def kernel(x, lut):
    idx2 = x.reshape(B_TOTAL // 128, 128).astype(jnp.int32)
    out = _embed_gather(lut, idx2)
    return out.reshape(x.shape[0], x.shape[1], D_MODEL)


# speedup vs baseline: 1.3201x; 1.1608x over previous
import functools
import math

import jax
import jax.numpy as jnp
from jax import lax
from jax.experimental import pallas as pl
from jax.experimental.pallas import tpu as pltpu
from jax.experimental.pallas import tpu_sc as plsc

D_MODEL = 64
D_PAD = 128
VOCAB = 1000000
B_TOTAL = 4096 * 200
NC, NS = 2, 16
NW = NC * NS
ROWS_PER_W = B_TOTAL // NW
SUB = 1
CHUNK = SUB * 128
N_CHUNKS = ROWS_PER_W // CHUNK
NBUF = 2
SCALE = math.sqrt(D_MODEL)

_mesh = plsc.VectorSubcoreMesh(
    core_axis_name="c", subcore_axis_name="s", num_cores=NC, num_subcores=NS
)


@functools.partial(
    pl.kernel,
    out_type=jax.ShapeDtypeStruct((B_TOTAL, D_MODEL), jnp.float32),
    mesh=_mesh,
    scratch_types=[
        pltpu.VMEM((NBUF, SUB, 128), jnp.int32),
        pltpu.VMEM((NBUF, CHUNK, D_PAD), jnp.float32),
        pltpu.VMEM((NBUF, CHUNK, D_MODEL), jnp.float32),
        pltpu.SemaphoreType.DMA,
        pltpu.SemaphoreType.DMA,
        pltpu.SemaphoreType.DMA,
        pltpu.SemaphoreType.DMA,
        pltpu.SemaphoreType.DMA,
        pltpu.SemaphoreType.DMA,
    ],
    compiler_params=pltpu.CompilerParams(use_tc_tiling_on_sc=True),
)
def _embed_gather(lut_hbm, idx_hbm, out_hbm, idx_v, rows_v, outs_v,
                  si0, si1, sg0, sg1, ss0, ss1):
    si = (si0, si1)
    sg = (sg0, sg1)
    ss = (ss0, ss1)
    wid = lax.axis_index("s") * NC + lax.axis_index("c")
    irow0 = wid * (ROWS_PER_W // 128)
    row0 = wid * ROWS_PER_W

    def idx_start(g, b):
        pltpu.async_copy(
            idx_hbm.at[pl.ds(irow0 + g * SUB, SUB)], idx_v.at[b], si[b])

    def idx_wait(b):
        pltpu.make_async_copy(
            idx_hbm.at[pl.ds(irow0, SUB)], idx_v.at[b], si[b]).wait()

    def gat_start(b):
        for j in range(SUB):
            pltpu.async_copy(
                lut_hbm.at[idx_v.at[b, j]],
                rows_v.at[b, pl.ds(j * 128, 128)], sg[b])

    def gat_wait(b):
        for j in range(SUB):
            pltpu.make_async_copy(
                lut_hbm.at[idx_v.at[b, j]],
                rows_v.at[b, pl.ds(j * 128, 128)], sg[b]).wait()

    def store_start(g, b):
        pltpu.async_copy(
            outs_v.at[b], out_hbm.at[pl.ds(row0 + g * CHUNK, CHUNK)], ss[b])

    def store_wait(b):
        pltpu.make_async_copy(
            outs_v.at[b], out_hbm.at[pl.ds(row0, CHUNK)], ss[b]).wait()

    def scale_buf(b):
        rv = rows_v.at[b]
        ov = outs_v.at[b]

        def scale_blk(k, c):
            base = k * 4
            for u in range(4):
                for j in range(D_MODEL // 16):
                    sl = pl.ds(j * 16, 16)
                    ov[base + u, sl] = rv[base + u, sl] * SCALE
            return c

        lax.fori_loop(0, CHUNK // 4, scale_blk, 0)

    idx_start(0, 0)
    idx_start(1, 1)
    idx_wait(0)
    gat_start(0)

    def step(it, carry):
        for b in range(NBUF):
            g = it * NBUF + b
            o = b ^ 1
            gat_wait(b)
            pl.when(g < N_CHUNKS - NBUF)(lambda: idx_start(g + NBUF, b))
            pl.when(g >= 1)(lambda: store_wait(o))
            def fire_next():
                idx_wait(o)
                gat_start(o)
            pl.when(g < N_CHUNKS - 1)(fire_next)
            scale_buf(b)
            store_start(g, b)
        return carry

    lax.fori_loop(0, N_CHUNKS // NBUF, step, 0)

    store_wait((N_CHUNKS - 1) % NBUF)


def kernel(x, lut):
    lut_p = jnp.pad(lut, ((0, 0), (0, D_PAD - D_MODEL)))
    idx2 = x.reshape(B_TOTAL // 128, 128).astype(jnp.int32)
    out = _embed_gather(lut_p, idx2)
    return out.reshape(x.shape[0], x.shape[1], D_MODEL)
